# baseline (speedup 1.0000x reference)
import jax
import jax.numpy as jnp
from jax import lax
from jax.experimental import pallas as pl
from jax.experimental.pallas import tpu as pltpu
from jax.experimental.pallas import tpu_sc as plsc

N = 10000
E = 320000
D = 128
W = 136
NCLS = 64

NC = 2
NS = 16
NW = NC * NS
EPT = E // NW
K = 32
CHUNKS = 313
EPTP = CHUNKS * K
G = K // 16
RPT = N // NS

NB = 3
NBI = 8


_R = 1000


def _fc1_table_body(x_ref, w1_ref, b1_ref, out_ref):
    x = lax.dot_general(x_ref[...], w1_ref[...],
                        dimension_numbers=(((1,), (1,)), ((), ())),
                        preferred_element_type=jnp.float32)
    x = jnp.maximum(x + b1_ref[...], 0.0)
    nc = jnp.maximum(jnp.sqrt(jnp.sum(x * x, axis=1, keepdims=True)), 1e-12)
    out_ref[:, 0:D] = x / nc
    cols = lax.broadcasted_iota(jnp.int32, (_R, W - D), 1)
    out_ref[:, D:W] = jnp.where(cols == 0, nc, 0.0)


def _fc1_table(x, w1, b1):
    return pl.pallas_call(
        _fc1_table_body,
        grid=(N // _R,),
        in_specs=[
            pl.BlockSpec((_R, D), lambda i: (i, 0)),
            pl.BlockSpec((D, D), lambda i: (0, 0)),
            pl.BlockSpec((D,), lambda i: (0,)),
        ],
        out_specs=pl.BlockSpec((_R, W), lambda i: (i, 0)),
        out_shape=jax.ShapeDtypeStruct((N, W), jnp.float32),
    )(x, w1, b1)


def _combine_table_body(p_ref, out_ref):
    row = p_ref[0] + p_ref[1]
    s = row[:, D:D + 1]
    x1 = row[:, 0:D] / (s + 1e-12)
    nc = jnp.maximum(jnp.sqrt(jnp.sum(x1 * x1, axis=1, keepdims=True)), 1e-12)
    out_ref[:, 0:D] = x1 / nc
    cols = lax.broadcasted_iota(jnp.int32, (_R, W - D), 1)
    out_ref[:, D:W] = jnp.where(cols == 0, nc, 0.0)


def _combine_table(p):
    return pl.pallas_call(
        _combine_table_body,
        grid=(N // _R,),
        in_specs=[pl.BlockSpec((2, _R, W), lambda i: (0, i, 0))],
        out_specs=pl.BlockSpec((_R, W), lambda i: (i, 0)),
        out_shape=jax.ShapeDtypeStruct((N, W), jnp.float32),
    )(p)


def _final_body(p_ref, w2_ref, b2_ref, out_ref):
    row = p_ref[0] + p_ref[1]
    s = row[:, D:D + 1]
    x2 = row[:, 0:D] / (s + 1e-12)
    y = lax.dot_general(x2, w2_ref[...],
                        dimension_numbers=(((1,), (1,)), ((), ())),
                        preferred_element_type=jnp.float32)
    out_ref[...] = y + b2_ref[...]


def _final(p, w2, b2):
    return pl.pallas_call(
        _final_body,
        grid=(N // _R,),
        in_specs=[
            pl.BlockSpec((2, _R, W), lambda i: (0, i, 0)),
            pl.BlockSpec((NCLS, D), lambda i: (0, 0)),
            pl.BlockSpec((NCLS,), lambda i: (0,)),
        ],
        out_specs=pl.BlockSpec((_R, NCLS), lambda i: (i, 0)),
        out_shape=jax.ShapeDtypeStruct((N, NCLS), jnp.float32),
    )(p, w2, b2)


def _agnn_sc_body(table_hbm, src_hbm, dst_hbm, beta_hbm, out_hbm,
                  src_i, dst_i, sidx, src_r, dst_r, msg_r,
                  beta_v, acc_sh, isem, gsem, ssem):
    c = lax.axis_index("c")
    s = lax.axis_index("s")
    wid = c * NS + s

    pltpu.sync_copy(beta_hbm, beta_v)

    zv = jnp.zeros((16,), jnp.float32)
    lanes = lax.iota(jnp.int32, 16)
    bvec = beta_v[...]

    def zero_msgs(r, carry):
        for b in range(NB):
            for cc in range(W // 16):
                msg_r[b, r, pl.ds(cc * 16, 16)] = zv
            msg_r[b, r, pl.ds(W - 16, 16)] = zv
        return carry

    lax.fori_loop(0, K, zero_msgs, 0)
    r0 = s * RPT
    for t in range(RPT // K):
        pltpu.sync_copy(msg_r.at[0], acc_sh.at[pl.ds(r0 + t * K, K)])
    pltpu.sync_copy(msg_r.at[0].at[pl.ds(0, RPT - (RPT // K) * K)],
                    acc_sh.at[pl.ds(r0 + (RPT // K) * K, RPT - (RPT // K) * K)])
    plsc.subcore_barrier()

    for j0 in range(2):
        pltpu.sync_copy(src_hbm.at[wid, j0], src_i.at[j0])
        pltpu.sync_copy(dst_hbm.at[wid, j0], dst_i.at[j0])
    for j0 in range(2, 4):
        pltpu.async_copy(src_hbm.at[wid, j0], src_i.at[j0], isem)
        pltpu.async_copy(dst_hbm.at[wid, j0], dst_i.at[j0], isem)
    for j0 in range(2):
        pltpu.async_copy(table_hbm.at[src_i.at[j0]], src_r.at[j0], gsem)
        pltpu.async_copy(table_hbm.at[dst_i.at[j0]], dst_r.at[j0], gsem)

    def chunk_body(j, carry):
        slot = jax.lax.rem(j, NB)
        slotv = jnp.full((16,), slot, jnp.int32)
        islot = jax.lax.rem(j, NBI)

        @pl.when(j >= NB)
        def _():
            pltpu.make_async_copy(table_hbm.at[src_i.at[0]], msg_r.at[slot],
                                  ssem).wait()

        pltpu.make_async_copy(table_hbm.at[src_i.at[0]], src_r.at[slot],
                              gsem).wait()
        pltpu.make_async_copy(table_hbm.at[src_i.at[0]], dst_r.at[slot],
                              gsem).wait()

        @pl.when(j + 2 < CHUNKS)
        def _():
            i2 = jax.lax.rem(j + 2, NBI)
            b2 = jax.lax.rem(j + 2, NB)
            pltpu.make_async_copy(src_hbm.at[wid, 0], src_i.at[0], isem).wait()
            pltpu.make_async_copy(dst_hbm.at[wid, 0], dst_i.at[0], isem).wait()
            pltpu.async_copy(table_hbm.at[src_i.at[i2]], src_r.at[b2], gsem)
            pltpu.async_copy(table_hbm.at[dst_i.at[i2]], dst_r.at[b2], gsem)

        @pl.when(j + 4 < CHUNKS)
        def _():
            i4 = jax.lax.rem(j + 4, NBI)
            pltpu.async_copy(src_hbm.at[wid, j + 4], src_i.at[i4], isem)
            pltpu.async_copy(dst_hbm.at[wid, j + 4], dst_i.at[i4], isem)

        for g in range(G):
            rows16 = g * 16 + lanes
            nrm = plsc.load_gather(src_r, [slotv, rows16,
                                           jnp.full((16,), D, jnp.int32)])

            def dot_body(d, a0):
                col = jnp.full((16,), d, jnp.int32)
                a = plsc.load_gather(src_r, [slotv, rows16, col])
                b = plsc.load_gather(dst_r, [slotv, rows16, col])
                return a0 + a * b

            acc = plsc.parallel_loop(
                0, D, unroll=8,
                carry=jnp.zeros((16,), jnp.float32))(dot_body)
            ex = jnp.exp(bvec * acc)
            kv = ex * nrm
            valid = (j * K + g * 16 + lanes) < EPT
            ex = jnp.where(valid, ex, 0.0)
            kv = jnp.where(valid, kv, 0.0)

            @plsc.parallel_loop(0, D, unroll=8)
            def scale_body(d):
                col = jnp.full((16,), d, jnp.int32)
                v = plsc.load_gather(src_r, [slotv, rows16, col]) * kv
                plsc.store_scatter(msg_r, [slotv, rows16, col], v)

            plsc.store_scatter(
                msg_r, [slotv, rows16, jnp.full((16,), D, jnp.int32)], ex)

            sidx[slot, pl.ds(g * 16, 16)] = dst_i[islot, pl.ds(g * 16, 16)]

        pltpu.async_copy(msg_r.at[slot], acc_sh.at[sidx.at[slot]], ssem,
                         add=True)
        return carry

    lax.fori_loop(0, CHUNKS, chunk_body, 0)

    for b in range(NB):
        pltpu.make_async_copy(table_hbm.at[src_i.at[0]], msg_r.at[b],
                              ssem).wait()
    plsc.subcore_barrier()

    pltpu.sync_copy(acc_sh.at[pl.ds(r0, RPT)],
                    out_hbm.at[c, pl.ds(r0, RPT)])


def _agnn_layer(table, src4, dst4, beta_arr):
    mesh = plsc.VectorSubcoreMesh(core_axis_name="c", subcore_axis_name="s",
                                  num_cores=NC, num_subcores=NS)
    f = pl.kernel(
        _agnn_sc_body,
        out_type=jax.ShapeDtypeStruct((NC, N, W), jnp.float32),
        mesh=mesh,
        scratch_types=[
            pltpu.VMEM((NBI, K), jnp.int32),
            pltpu.VMEM((NBI, K), jnp.int32),
            pltpu.VMEM((NB, K), jnp.int32),
            pltpu.VMEM((NB, K, W), jnp.float32),
            pltpu.VMEM((NB, K, W), jnp.float32),
            pltpu.VMEM((NB, K, W), jnp.float32),
            pltpu.VMEM((16,), jnp.float32),
            pltpu.VMEM_SHARED((N, W), jnp.float32),
            pltpu.SemaphoreType.DMA,
            pltpu.SemaphoreType.DMA,
            pltpu.SemaphoreType.DMA,
        ],
        compiler_params=pltpu.CompilerParams(use_tc_tiling_on_sc=False,
                                             needs_layout_passes=False),
    )
    return f(table, src4, dst4, beta_arr)


def kernel(input_features, edge_index, order_attn, W1, b1, beta1, beta2, W2, b2):
    ei = edge_index.reshape(2, NW, EPT)
    ei = jnp.pad(ei, ((0, 0), (0, 0), (0, EPTP - EPT)))
    src4 = ei[0].reshape(NW, CHUNKS, K)
    dst4 = ei[1].reshape(NW, CHUNKS, K)
    beta1_arr = jnp.full((16,), beta1, jnp.float32)
    beta2_arr = jnp.full((16,), beta2, jnp.float32)

    table0 = _fc1_table(input_features, W1, b1)
    p1 = _agnn_layer(table0, src4, dst4, beta1_arr)
    table1 = _combine_table(p1)
    p2 = _agnn_layer(table1, src4, dst4, beta2_arr)
    return _final(p2, W2, b2)


# speedup vs baseline: 1.0642x; 1.0642x over previous
import jax
import jax.numpy as jnp
from jax import lax
from jax.experimental import pallas as pl
from jax.experimental.pallas import tpu as pltpu
from jax.experimental.pallas import tpu_sc as plsc

N = 10000
E = 320000
D = 128
W = 136
NCLS = 64

NC = 2
NS = 16
NW = NC * NS
EPT = E // NW
K = 16
CHUNKS = EPT // K
RPT = N // NS

NB = 6
NBI = 8


_R = 1000


def _fc1_table_body(x_ref, w1_ref, b1_ref, out_ref):
    x = lax.dot_general(x_ref[...], w1_ref[...],
                        dimension_numbers=(((1,), (1,)), ((), ())),
                        preferred_element_type=jnp.float32)
    x = jnp.maximum(x + b1_ref[...], 0.0)
    nc = jnp.maximum(jnp.sqrt(jnp.sum(x * x, axis=1, keepdims=True)), 1e-12)
    out_ref[:, 0:D] = x / nc
    cols = lax.broadcasted_iota(jnp.int32, (_R, W - D), 1)
    out_ref[:, D:W] = jnp.where(cols == 0, nc, 0.0)


def _fc1_table(x, w1, b1):
    return pl.pallas_call(
        _fc1_table_body,
        grid=(N // _R,),
        in_specs=[
            pl.BlockSpec((_R, D), lambda i: (i, 0)),
            pl.BlockSpec((D, D), lambda i: (0, 0)),
            pl.BlockSpec((D,), lambda i: (0,)),
        ],
        out_specs=pl.BlockSpec((_R, W), lambda i: (i, 0)),
        out_shape=jax.ShapeDtypeStruct((N, W), jnp.float32),
    )(x, w1, b1)


def _combine_table_body(p_ref, out_ref):
    row = p_ref[0] + p_ref[1]
    s = row[:, D:D + 1]
    x1 = row[:, 0:D] / (s + 1e-12)
    nc = jnp.maximum(jnp.sqrt(jnp.sum(x1 * x1, axis=1, keepdims=True)), 1e-12)
    out_ref[:, 0:D] = x1 / nc
    cols = lax.broadcasted_iota(jnp.int32, (_R, W - D), 1)
    out_ref[:, D:W] = jnp.where(cols == 0, nc, 0.0)


def _combine_table(p):
    return pl.pallas_call(
        _combine_table_body,
        grid=(N // _R,),
        in_specs=[pl.BlockSpec((2, _R, W), lambda i: (0, i, 0))],
        out_specs=pl.BlockSpec((_R, W), lambda i: (i, 0)),
        out_shape=jax.ShapeDtypeStruct((N, W), jnp.float32),
    )(p)


def _final_body(p_ref, w2_ref, b2_ref, out_ref):
    row = p_ref[0] + p_ref[1]
    s = row[:, D:D + 1]
    x2 = row[:, 0:D] / (s + 1e-12)
    y = lax.dot_general(x2, w2_ref[...],
                        dimension_numbers=(((1,), (1,)), ((), ())),
                        preferred_element_type=jnp.float32)
    out_ref[...] = y + b2_ref[...]


def _final(p, w2, b2):
    return pl.pallas_call(
        _final_body,
        grid=(N // _R,),
        in_specs=[
            pl.BlockSpec((2, _R, W), lambda i: (0, i, 0)),
            pl.BlockSpec((NCLS, D), lambda i: (0, 0)),
            pl.BlockSpec((NCLS,), lambda i: (0,)),
        ],
        out_specs=pl.BlockSpec((_R, NCLS), lambda i: (i, 0)),
        out_shape=jax.ShapeDtypeStruct((N, NCLS), jnp.float32),
    )(p, w2, b2)


def _agnn_sc_body(table_hbm, src_hbm, dst_hbm, beta_hbm, out_hbm,
                  src_i, dst_i, sidx, src_r, dst_r, msg_r,
                  beta_v, acc_sh, isem, gsem, ssem):
    c = lax.axis_index("c")
    s = lax.axis_index("s")
    wid = c * NS + s

    pltpu.sync_copy(beta_hbm, beta_v)

    zv = jnp.zeros((16,), jnp.float32)
    lanes = lax.iota(jnp.int32, 16)
    bvec = beta_v[...]

    def zero_msgs(r, carry):
        for b in range(NB):
            for cc in range(W // 16):
                msg_r[b, r, pl.ds(cc * 16, 16)] = zv
            msg_r[b, r, pl.ds(W - 16, 16)] = zv
        return carry

    lax.fori_loop(0, K, zero_msgs, 0)
    r0 = s * RPT
    for t in range(RPT // K):
        pltpu.sync_copy(msg_r.at[0], acc_sh.at[pl.ds(r0 + t * K, K)])
    pltpu.sync_copy(msg_r.at[0].at[pl.ds(0, RPT - (RPT // K) * K)],
                    acc_sh.at[pl.ds(r0 + (RPT // K) * K, RPT - (RPT // K) * K)])
    plsc.subcore_barrier()

    for j0 in range(4):
        pltpu.sync_copy(src_hbm.at[wid, j0], src_i.at[j0])
        pltpu.sync_copy(dst_hbm.at[wid, j0], dst_i.at[j0])
    for j0 in range(4, 7):
        pltpu.async_copy(src_hbm.at[wid, j0], src_i.at[j0], isem)
        pltpu.async_copy(dst_hbm.at[wid, j0], dst_i.at[j0], isem)
    for j0 in range(4):
        pltpu.async_copy(table_hbm.at[src_i.at[j0]], src_r.at[j0], gsem)
        pltpu.async_copy(table_hbm.at[dst_i.at[j0]], dst_r.at[j0], gsem)

    def chunk_body(j, carry):
        slot = jax.lax.rem(j, NB)
        slotv = jnp.full((16,), slot, jnp.int32)
        islot = jax.lax.rem(j, NBI)

        @pl.when(j >= NB)
        def _():
            pltpu.make_async_copy(table_hbm.at[src_i.at[0]], msg_r.at[slot],
                                  ssem).wait()

        pltpu.make_async_copy(table_hbm.at[src_i.at[0]], src_r.at[slot],
                              gsem).wait()
        pltpu.make_async_copy(table_hbm.at[src_i.at[0]], dst_r.at[slot],
                              gsem).wait()

        @pl.when(j + 4 < CHUNKS)
        def _():
            i4 = jax.lax.rem(j + 4, NBI)
            b4 = jax.lax.rem(j + 4, NB)
            pltpu.make_async_copy(src_hbm.at[wid, 0], src_i.at[0], isem).wait()
            pltpu.make_async_copy(dst_hbm.at[wid, 0], dst_i.at[0], isem).wait()
            pltpu.async_copy(table_hbm.at[src_i.at[i4]], src_r.at[b4], gsem)
            pltpu.async_copy(table_hbm.at[dst_i.at[i4]], dst_r.at[b4], gsem)

        @pl.when(j + 7 < CHUNKS)
        def _():
            i7 = jax.lax.rem(j + 7, NBI)
            pltpu.async_copy(src_hbm.at[wid, j + 7], src_i.at[i7], isem)
            pltpu.async_copy(dst_hbm.at[wid, j + 7], dst_i.at[i7], isem)

        nrm = plsc.load_gather(src_r, [slotv, lanes,
                                       jnp.full((16,), D, jnp.int32)])

        def dot_body(d, a0):
            col = jnp.full((16,), d, jnp.int32)
            a = plsc.load_gather(src_r, [slotv, lanes, col])
            b = plsc.load_gather(dst_r, [slotv, lanes, col])
            return a0 + a * b

        acc = plsc.parallel_loop(0, D, unroll=8,
                                 carry=jnp.zeros((16,), jnp.float32))(dot_body)
        ex = jnp.exp(bvec * acc)
        kv = ex * nrm

        @plsc.parallel_loop(0, D, unroll=8)
        def scale_body(d):
            col = jnp.full((16,), d, jnp.int32)
            v = plsc.load_gather(src_r, [slotv, lanes, col]) * kv
            plsc.store_scatter(msg_r, [slotv, lanes, col], v)

        plsc.store_scatter(msg_r, [slotv, lanes, jnp.full((16,), D, jnp.int32)],
                           ex)

        sidx[slot, pl.ds(0, 16)] = dst_i[islot, pl.ds(0, 16)]

        pltpu.async_copy(msg_r.at[slot], acc_sh.at[sidx.at[slot]], ssem,
                         add=True)
        return carry

    lax.fori_loop(0, CHUNKS, chunk_body, 0)

    for b in range(NB):
        pltpu.make_async_copy(table_hbm.at[src_i.at[0]], msg_r.at[b],
                              ssem).wait()
    plsc.subcore_barrier()

    pltpu.sync_copy(acc_sh.at[pl.ds(r0, RPT)],
                    out_hbm.at[c, pl.ds(r0, RPT)])


def _agnn_layer(table, src4, dst4, beta_arr):
    mesh = plsc.VectorSubcoreMesh(core_axis_name="c", subcore_axis_name="s",
                                  num_cores=NC, num_subcores=NS)
    f = pl.kernel(
        _agnn_sc_body,
        out_type=jax.ShapeDtypeStruct((NC, N, W), jnp.float32),
        mesh=mesh,
        scratch_types=[
            pltpu.VMEM((NBI, K), jnp.int32),
            pltpu.VMEM((NBI, K), jnp.int32),
            pltpu.VMEM((NB, K), jnp.int32),
            pltpu.VMEM((NB, K, W), jnp.float32),
            pltpu.VMEM((NB, K, W), jnp.float32),
            pltpu.VMEM((NB, K, W), jnp.float32),
            pltpu.VMEM((16,), jnp.float32),
            pltpu.VMEM_SHARED((N, W), jnp.float32),
            pltpu.SemaphoreType.DMA,
            pltpu.SemaphoreType.DMA,
            pltpu.SemaphoreType.DMA,
        ],
        compiler_params=pltpu.CompilerParams(use_tc_tiling_on_sc=False,
                                             needs_layout_passes=False),
    )
    return f(table, src4, dst4, beta_arr)


def kernel(input_features, edge_index, order_attn, W1, b1, beta1, beta2, W2, b2):
    src4 = edge_index[0].reshape(NW, CHUNKS, K)
    dst4 = edge_index[1].reshape(NW, CHUNKS, K)
    beta1_arr = jnp.full((16,), beta1, jnp.float32)
    beta2_arr = jnp.full((16,), beta2, jnp.float32)

    table0 = _fc1_table(input_features, W1, b1)
    p1 = _agnn_layer(table0, src4, dst4, beta1_arr)
    table1 = _combine_table(p1)
    p2 = _agnn_layer(table1, src4, dst4, beta2_arr)
    return _final(p2, W2, b2)
